# Initial kernel scaffold; baseline (speedup 1.0000x reference)
import jax
import jax.numpy as jnp
from jax.experimental import pallas as pl


def kernel(logits):
    raise NotImplementedError("write your pallas kernel here")


# speedup vs baseline: 1.8064x; 1.8064x over previous
import functools

import jax
import jax.numpy as jnp
from jax.experimental import pallas as pl

SUPPORT_RANGE = 300
EPS = 0.001
NUM_BINS = 2 * SUPPORT_RANGE + 1

BLOCK_ROWS = 512


def _mu_zero_block(logits_ref, out_ref):
    logits = logits_ref[...]
    rows = logits.shape[0]

    m = jnp.max(logits, axis=-1, keepdims=True)
    e = jnp.exp(logits - m)
    denom = jnp.sum(e, axis=-1)
    bins = jax.lax.broadcasted_iota(jnp.int32, (rows, NUM_BINS), 1)
    support = bins.astype(jnp.float32) - float(SUPPORT_RANGE)
    num = jnp.sum(e * support, axis=-1)
    x = num / denom

    scalar = jnp.sign(x) * (
        ((jnp.sqrt(1.0 + 4.0 * EPS * (jnp.abs(x) + 1.0 + EPS)) - 1.0) / (2.0 * EPS))
        ** 2
        - 1.0
    )

    y = jnp.sign(scalar) * (jnp.sqrt(jnp.abs(scalar) + 1.0) - 1.0) + EPS * scalar
    y = jnp.clip(y, -float(SUPPORT_RANGE), float(SUPPORT_RANGE))
    floor = jnp.floor(y)
    prob = y - floor
    idx_low = jnp.clip(
        (floor + float(SUPPORT_RANGE)).astype(jnp.int32), 0, NUM_BINS - 1
    )
    idx_high = jnp.clip(idx_low + 1, 0, NUM_BINS - 1)

    low_hit = (bins == idx_low[:, None]).astype(jnp.float32)
    high_hit = (bins == idx_high[:, None]).astype(jnp.float32)
    out_ref[...] = low_hit * (1.0 - prob[:, None]) + high_hit * prob[:, None]


@jax.jit
def kernel(logits):
    n_rows = logits.shape[0]
    grid = (n_rows // BLOCK_ROWS,)
    return pl.pallas_call(
        _mu_zero_block,
        grid=grid,
        in_specs=[pl.BlockSpec((BLOCK_ROWS, NUM_BINS), lambda i: (i, 0))],
        out_specs=pl.BlockSpec((BLOCK_ROWS, NUM_BINS), lambda i: (i, 0)),
        out_shape=jax.ShapeDtypeStruct((n_rows, NUM_BINS), jnp.float32),
    )(logits)
